# Initial kernel scaffold; baseline (speedup 1.0000x reference)
import jax
import jax.numpy as jnp
from jax.experimental import pallas as pl


def kernel(comment_centers, code_centers, all_code_centers, comment_to_code_map, negative_sample_indices, nl_hidden, code_hidden, total_code_tokens_list, valid_code_spans_batch, valid_comment_spans_batch, step_descriptions_batch):
    raise NotImplementedError("write your pallas kernel here")


# speedup vs baseline: 180.4399x; 180.4399x over previous
import functools

import jax
import jax.numpy as jnp
from jax.experimental import pallas as pl

TEMPERATURE = 0.1


def _pool_kernel(starts_ref, lims_ref, ch_ref, out_ref, cnt_ref):
    s = starts_ref[0, 0, :]
    lim = lims_ref[0, 0, :]
    C = s.shape[0]
    L = ch_ref.shape[1]
    t = jax.lax.broadcasted_iota(jnp.int32, (C, L), 1)
    mask = (t >= s[:, None]) & (t <= lim[:, None])
    maskf = mask.astype(jnp.float32)
    out_ref[0] = jnp.dot(maskf, ch_ref[0], preferred_element_type=jnp.float32)
    cnt_ref[0, 0, :] = jnp.sum(mask, axis=1).astype(jnp.int32)


def _loss_kernel(cc_ref, codec_ref, c2c_ref, nb_ref, ns_ref, pooled_ref,
                 cnt_ref, out_ref):
    N = cc_ref.shape[0]
    K = nb_ref.shape[2]
    eps = jnp.float32(1e-12)

    cc = cc_ref[...]
    cc = cc / jnp.maximum(
        jnp.sqrt(jnp.sum(cc * cc, axis=1, keepdims=True)), eps)
    codec = codec_ref[...]
    codec = codec / jnp.maximum(
        jnp.sqrt(jnp.sum(codec * codec, axis=1, keepdims=True)), eps)

    c2c = c2c_ref[0, 0, :]
    c2c_cl = jnp.clip(c2c, 0, N - 1)
    jj = jax.lax.broadcasted_iota(jnp.int32, (N, N), 1)
    sel_pos = (jj == c2c_cl[:, None]).astype(jnp.float32)
    code_cent = jnp.dot(sel_pos, codec, preferred_element_type=jnp.float32)
    pos_sim = jnp.sum(cc * code_cent, axis=1)

    cnt = cnt_ref[0, 0, :].astype(jnp.float32)
    pooled = pooled_ref[...]
    pooled = pooled / jnp.maximum(cnt, 1.0)[:, None]
    pooled = pooled / jnp.maximum(
        jnp.sqrt(jnp.sum(pooled * pooled, axis=1, keepdims=True)), eps)
    S = jnp.dot(cc, pooled.T, preferred_element_type=jnp.float32)

    nb = nb_ref[0, :, :]
    ns = ns_ref[0, :, :]
    B = 8
    C = N // B
    in_range = (nb < B) & (ns < C)
    j = jnp.clip(nb, 0, B - 1) * C + jnp.clip(ns, 0, C - 1)
    jk = jax.lax.broadcasted_iota(jnp.int32, (N, K, N), 2)
    sel = (jk == j[:, :, None]).astype(jnp.float32)
    E = jnp.sum(S[:, None, :] * sel, axis=2)
    cnt_pos = (cnt > 0.0).astype(jnp.float32)
    neg_has = jnp.sum(cnt_pos[None, None, :] * sel, axis=2) > 0.0
    vmask = in_range & neg_has

    neg_exp = jnp.exp(E / TEMPERATURE)
    neg_sum = jnp.sum(jnp.where(vmask, neg_exp, 0.0), axis=1)
    pos_exp = jnp.exp(pos_sim / TEMPERATURE)
    lv = -jnp.log(pos_exp / (pos_exp + neg_sum + 1e-08))
    valid = (c2c < N) & jnp.any(vmask, axis=1)
    vals = jnp.where(valid, lv, 0.0)
    total = jnp.sum(vals)
    n = jnp.sum(valid.astype(jnp.float32))
    res = jnp.where(n > 0.0, total / jnp.maximum(n, 1.0), 0.0)
    out_ref[...] = jnp.reshape(res, (1, 1))


@jax.jit
def kernel(comment_centers, code_centers, all_code_centers,
           comment_to_code_map, negative_sample_indices, nl_hidden,
           code_hidden, total_code_tokens_list, valid_code_spans_batch,
           valid_comment_spans_batch, step_descriptions_batch):
    del all_code_centers, nl_hidden, valid_comment_spans_batch
    del step_descriptions_batch
    B, L, H = code_hidden.shape
    N, _ = comment_centers.shape
    Bn, C, K, _ = negative_sample_indices.shape

    spans = valid_code_spans_batch.astype(jnp.int32)
    starts = spans[:, :, 1, 0].reshape(B, 1, C)
    totals = total_code_tokens_list.astype(jnp.int32)
    lims = jnp.minimum(spans[:, :, 1, 1],
                       totals[:, None]).reshape(B, 1, C)

    pooled, cnt = pl.pallas_call(
        _pool_kernel,
        grid=(B,),
        in_specs=[
            pl.BlockSpec((1, 1, C), lambda b: (b, 0, 0)),
            pl.BlockSpec((1, 1, C), lambda b: (b, 0, 0)),
            pl.BlockSpec((1, L, H), lambda b: (b, 0, 0)),
        ],
        out_specs=[
            pl.BlockSpec((1, C, H), lambda b: (b, 0, 0)),
            pl.BlockSpec((1, 1, C), lambda b: (b, 0, 0)),
        ],
        out_shape=[
            jax.ShapeDtypeStruct((B, C, H), jnp.float32),
            jax.ShapeDtypeStruct((B, 1, C), jnp.int32),
        ],
    )(starts, lims, code_hidden)

    negs = negative_sample_indices.astype(jnp.int32).reshape(N, K, 2)
    nb = negs[:, :, 0].reshape(1, N, K)
    ns = negs[:, :, 1].reshape(1, N, K)
    c2c = comment_to_code_map.astype(jnp.int32).reshape(1, 1, N)

    out = pl.pallas_call(
        _loss_kernel,
        in_specs=[
            pl.BlockSpec((N, H), lambda: (0, 0)),
            pl.BlockSpec((N, H), lambda: (0, 0)),
            pl.BlockSpec((1, 1, N), lambda: (0, 0, 0)),
            pl.BlockSpec((1, N, K), lambda: (0, 0, 0)),
            pl.BlockSpec((1, N, K), lambda: (0, 0, 0)),
            pl.BlockSpec((N, H), lambda: (0, 0)),
            pl.BlockSpec((1, 1, N), lambda: (0, 0, 0)),
        ],
        out_specs=pl.BlockSpec((1, 1), lambda: (0, 0)),
        out_shape=jax.ShapeDtypeStruct((1, 1), jnp.float32),
    )(comment_centers, code_centers, c2c, nb, ns,
      pooled.reshape(N, H), cnt.reshape(1, 1, N))

    return out[0, 0]


# speedup vs baseline: 191.9912x; 1.0640x over previous
import functools

import jax
import jax.numpy as jnp
from jax.experimental import pallas as pl
from jax.experimental.pallas import tpu as pltpu

TEMPERATURE = 0.1


def _fused_kernel(starts_ref, lims_ref, ch_ref, cc_ref, codec_ref, c2c_ref,
                  nb_ref, ns_ref, sall_ref, lall_ref, out_ref, pooled_ref,
                  *, B, C, K, N):
    b = pl.program_id(0)
    s = starts_ref[0, 0, :]
    lim = lims_ref[0, 0, :]
    L = ch_ref.shape[1]
    t = jax.lax.broadcasted_iota(jnp.int32, (C, L), 1)
    mask = (t >= s[:, None]) & (t <= lim[:, None])
    maskf = mask.astype(jnp.float32)
    pooled_ref[pl.ds(b * C, C), :] = jnp.dot(
        maskf, ch_ref[0], preferred_element_type=jnp.float32)

    @pl.when(b == B - 1)
    def _epilogue():
        eps = jnp.float32(1e-12)
        cc = cc_ref[...]
        cc = cc / jnp.maximum(
            jnp.sqrt(jnp.sum(cc * cc, axis=1, keepdims=True)), eps)
        codec = codec_ref[...]
        codec = codec / jnp.maximum(
            jnp.sqrt(jnp.sum(codec * codec, axis=1, keepdims=True)), eps)

        c2c = c2c_ref[0, 0, :]
        c2c_cl = jnp.clip(c2c, 0, N - 1)
        jj = jax.lax.broadcasted_iota(jnp.int32, (N, N), 1)
        sel_pos = (jj == c2c_cl[:, None]).astype(jnp.float32)
        code_cent = jnp.dot(sel_pos, codec,
                            preferred_element_type=jnp.float32)
        pos_sim = jnp.sum(cc * code_cent, axis=1)

        cnt = jnp.maximum(
            lall_ref[0, 0, :] - sall_ref[0, 0, :] + 1, 0
        ).astype(jnp.float32)
        pooled = pooled_ref[...]
        pooled = pooled / jnp.maximum(cnt, 1.0)[:, None]
        pooled = pooled / jnp.maximum(
            jnp.sqrt(jnp.sum(pooled * pooled, axis=1, keepdims=True)), eps)
        S = jnp.dot(cc, pooled.T,
                    preferred_element_type=jnp.float32)

        nb = nb_ref[0, :, :]
        ns = ns_ref[0, :, :]
        in_range = (nb < B) & (ns < C)
        j = jnp.clip(nb, 0, B - 1) * C + jnp.clip(ns, 0, C - 1)
        jk = jax.lax.broadcasted_iota(jnp.int32, (N, K, N), 2)
        sel = (jk == j[:, :, None]).astype(jnp.float32)
        E = jnp.sum(S[:, None, :] * sel, axis=2)
        cnt_pos = (cnt > 0.0).astype(jnp.float32)
        neg_has = jnp.sum(cnt_pos[None, None, :] * sel, axis=2) > 0.0
        vmask = in_range & neg_has

        neg_exp = jnp.exp(E / TEMPERATURE)
        neg_sum = jnp.sum(jnp.where(vmask, neg_exp, 0.0), axis=1)
        pos_exp = jnp.exp(pos_sim / TEMPERATURE)
        lv = -jnp.log(pos_exp / (pos_exp + neg_sum + 1e-08))
        valid = (c2c < N) & jnp.any(vmask, axis=1)
        vals = jnp.where(valid, lv, 0.0)
        total = jnp.sum(vals)
        n = jnp.sum(valid.astype(jnp.float32))
        res = jnp.where(n > 0.0, total / jnp.maximum(n, 1.0), 0.0)
        out_ref[...] = jnp.reshape(res, (1, 1))


@jax.jit
def kernel(comment_centers, code_centers, all_code_centers,
           comment_to_code_map, negative_sample_indices, nl_hidden,
           code_hidden, total_code_tokens_list, valid_code_spans_batch,
           valid_comment_spans_batch, step_descriptions_batch):
    del all_code_centers, nl_hidden, valid_comment_spans_batch
    del step_descriptions_batch
    B, L, H = code_hidden.shape
    N, _ = comment_centers.shape
    _, C, K, _ = negative_sample_indices.shape

    spans = valid_code_spans_batch.astype(jnp.int32)
    starts = spans[:, :, 1, 0].reshape(B, 1, C)
    totals = total_code_tokens_list.astype(jnp.int32)
    lims = jnp.minimum(spans[:, :, 1, 1],
                       totals[:, None]).reshape(B, 1, C)

    negs = negative_sample_indices.astype(jnp.int32).reshape(N, K, 2)
    nb = negs[:, :, 0].reshape(1, N, K)
    ns = negs[:, :, 1].reshape(1, N, K)
    c2c = comment_to_code_map.astype(jnp.int32).reshape(1, 1, N)

    out = pl.pallas_call(
        functools.partial(_fused_kernel, B=B, C=C, K=K, N=N),
        grid=(B,),
        in_specs=[
            pl.BlockSpec((1, 1, C), lambda b: (b, 0, 0)),
            pl.BlockSpec((1, 1, C), lambda b: (b, 0, 0)),
            pl.BlockSpec((1, L, H), lambda b: (b, 0, 0)),
            pl.BlockSpec((N, H), lambda b: (0, 0)),
            pl.BlockSpec((N, H), lambda b: (0, 0)),
            pl.BlockSpec((1, 1, N), lambda b: (0, 0, 0)),
            pl.BlockSpec((1, N, K), lambda b: (0, 0, 0)),
            pl.BlockSpec((1, N, K), lambda b: (0, 0, 0)),
            pl.BlockSpec((1, 1, N), lambda b: (0, 0, 0)),
            pl.BlockSpec((1, 1, N), lambda b: (0, 0, 0)),
        ],
        out_specs=pl.BlockSpec((1, 1), lambda b: (0, 0)),
        out_shape=jax.ShapeDtypeStruct((1, 1), jnp.float32),
        scratch_shapes=[
            pltpu.VMEM((N, H), jnp.float32),
        ],
    )(starts, lims, code_hidden, comment_centers, code_centers, c2c, nb, ns,
      starts.reshape(1, 1, N), lims.reshape(1, 1, N))

    return out[0, 0]
